# baseline (speedup 1.0000x reference)
import functools

import jax
import jax.numpy as jnp
from jax import lax
from jax.experimental import pallas as pl
from jax.experimental.pallas import tpu as pltpu
from jax.experimental.pallas import tpu_sc as plsc

B, N, L = 1024, 26, 20
VOCAB, D = 128, 128
TOT = B * N * L
NC, NS = 2, 16
NW = NC * NS
PER_W = TOT // NW
CHUNK = 128
NCHUNK = PER_W // CHUNK
NBUF = 5
GROUPS = NCHUNK // NBUF

_mesh = plsc.VectorSubcoreMesh(core_axis_name="c", subcore_axis_name="s")


@functools.partial(
    pl.kernel,
    mesh=_mesh,
    out_type=jax.ShapeDtypeStruct((TOT, D), jnp.float32),
    scratch_types=(
        [pltpu.VMEM((NCHUNK, CHUNK), jnp.int32)]
        + [pltpu.VMEM((CHUNK, D), jnp.float32) for _ in range(NBUF)]
        + [pltpu.SemaphoreType.DMA for _ in range(2 * NBUF)]
    ),
)
def _embed(table_hbm, idx_hbm, out_hbm, idx_v, *bufs_and_sems):
    rows = bufs_and_sems[:NBUF]
    sem_g = bufs_and_sems[NBUF:2 * NBUF]
    sem_s = bufs_and_sems[2 * NBUF:]
    wid = lax.axis_index("s") * NC + lax.axis_index("c")
    base = wid * PER_W

    def gather(chunk, b):
        pltpu.async_copy(table_hbm.at[idx_v.at[chunk]], rows[b], sem_g[b])

    def gather_wait(b):
        pltpu.make_async_copy(table_hbm.at[idx_v.at[0]], rows[b], sem_g[b]).wait()

    def store(chunk, b):
        dst = out_hbm.at[pl.ds(base + chunk * CHUNK, CHUNK)]
        pltpu.async_copy(rows[b], dst, sem_s[b])

    def store_wait(b):
        dst = out_hbm.at[pl.ds(base, CHUNK)]
        pltpu.make_async_copy(rows[b], dst, sem_s[b]).wait()

    pltpu.sync_copy(idx_hbm.at[wid], idx_v)
    for b in range(NBUF):
        gather(b, b)

    def body(g, carry):
        for b in range(NBUF):
            j = g * NBUF + b
            bp = (b - 1) % NBUF
            if b == 0:
                can_refill = g >= 1
            else:
                can_refill = g <= GROUPS - 2
            @pl.when(can_refill)
            def _refill(bp=bp, chunk=j + NBUF - 1):
                store_wait(bp)
                gather(chunk, bp)
            gather_wait(b)
            store(j, b)
        return carry

    lax.fori_loop(0, GROUPS, body, 0)
    for b in range(NBUF):
        store_wait(b)


def kernel(input_feature, table):
    idx = input_feature.reshape(NW, NCHUNK, CHUNK).astype(jnp.int32)
    out = _embed(table, idx)
    return out.reshape(B, N, L, D)


# speedup vs baseline: 2.0964x; 2.0964x over previous
import functools

import jax
import jax.numpy as jnp
from jax import lax
from jax.experimental import pallas as pl
from jax.experimental.pallas import tpu as pltpu
from jax.experimental.pallas import tpu_sc as plsc

B, N, L = 1024, 26, 20
VOCAB, D = 128, 128
TOT = B * N * L
NC, NS = 2, 16
NW = NC * NS
PER_W = TOT // NW
CHUNK = 128
NCHUNK = PER_W // CHUNK
NBUF = 5
GROUPS = NCHUNK // NBUF

_mesh = plsc.VectorSubcoreMesh(core_axis_name="c", subcore_axis_name="s")


@functools.partial(
    pl.kernel,
    mesh=_mesh,
    out_type=jax.ShapeDtypeStruct((TOT, D), jnp.float32),
    scratch_types=(
        [pltpu.VMEM((NCHUNK, CHUNK), jnp.int32)]
        + [pltpu.VMEM((CHUNK, D), jnp.float32) for _ in range(NBUF)]
        + [pltpu.SemaphoreType.DMA for _ in range(2 * NBUF)]
        + [pltpu.VMEM_SHARED((VOCAB, D), jnp.float32)]
    ),
)
def _embed(table_hbm, idx_hbm, out_hbm, idx_v, *bufs_and_sems):
    rows = bufs_and_sems[:NBUF]
    sem_g = bufs_and_sems[NBUF:2 * NBUF]
    sem_s = bufs_and_sems[2 * NBUF:3 * NBUF]
    table_sh = bufs_and_sems[3 * NBUF]
    wid = lax.axis_index("s") * NC + lax.axis_index("c")
    base = wid * PER_W

    @pl.when(lax.axis_index("s") == 0)
    def _stage():
        pltpu.sync_copy(table_hbm, rows[0])
        pltpu.sync_copy(rows[0], table_sh)

    plsc.subcore_barrier()

    def gather(chunk, b):
        pltpu.async_copy(table_sh.at[idx_v.at[chunk]], rows[b], sem_g[b])

    def gather_wait(b):
        pltpu.make_async_copy(table_sh.at[idx_v.at[0]], rows[b], sem_g[b]).wait()

    def store(chunk, b):
        dst = out_hbm.at[pl.ds(base + chunk * CHUNK, CHUNK)]
        pltpu.async_copy(rows[b], dst, sem_s[b])

    def store_wait(b):
        dst = out_hbm.at[pl.ds(base, CHUNK)]
        pltpu.make_async_copy(rows[b], dst, sem_s[b]).wait()

    pltpu.sync_copy(idx_hbm.at[wid], idx_v)
    for b in range(NBUF):
        gather(b, b)

    def body(g, carry):
        for b in range(NBUF):
            j = g * NBUF + b
            bp = (b - 1) % NBUF
            if b == 0:
                can_refill = g >= 1
            else:
                can_refill = g <= GROUPS - 2
            @pl.when(can_refill)
            def _refill(bp=bp, chunk=j + NBUF - 1):
                store_wait(bp)
                gather(chunk, bp)
            gather_wait(b)
            store(j, b)
        return carry

    lax.fori_loop(0, GROUPS, body, 0)
    for b in range(NBUF):
        store_wait(b)


def kernel(input_feature, table):
    idx = input_feature.reshape(NW, NCHUNK, CHUNK).astype(jnp.int32)
    out = _embed(table, idx)
    return out.reshape(B, N, L, D)


# speedup vs baseline: 10.6015x; 5.0569x over previous
import functools

import jax
import jax.numpy as jnp
from jax import lax
from jax.experimental import pallas as pl
from jax.experimental.pallas import tpu as pltpu
from jax.experimental.pallas import tpu_sc as plsc

B, N, L = 1024, 26, 20
VOCAB, D = 128, 128
TOT = B * N * L
NC, NS = 2, 16
NW = NC * NS
PER_W = TOT // NW
CHUNK = 128
NCHUNK = PER_W // CHUNK
NBUF = 5
GROUPS = NCHUNK // NBUF

_mesh = plsc.VectorSubcoreMesh(core_axis_name="c", subcore_axis_name="s")


@functools.partial(
    pl.kernel,
    mesh=_mesh,
    out_type=jax.ShapeDtypeStruct((TOT, D), jnp.float32),
    scratch_types=(
        [pltpu.VMEM((NCHUNK, CHUNK), jnp.int32)]
        + [pltpu.VMEM((CHUNK, D), jnp.float32) for _ in range(NBUF)]
        + [pltpu.SemaphoreType.DMA for _ in range(2 * NBUF)]
        + [pltpu.VMEM_SHARED((VOCAB, D), jnp.float32)]
    ),
)
def _embed(table_hbm, idx_hbm, out_hbm, idx_v, *bufs_and_sems):
    rows = bufs_and_sems[:NBUF]
    sem_g = bufs_and_sems[NBUF:2 * NBUF]
    sem_s = bufs_and_sems[2 * NBUF:3 * NBUF]
    table_sh = bufs_and_sems[3 * NBUF]
    wid = lax.axis_index("s") * NC + lax.axis_index("c")
    base = wid * PER_W

    @pl.when(lax.axis_index("s") == 0)
    def _stage():
        pltpu.sync_copy(table_hbm, rows[0])
        pltpu.sync_copy(rows[0], table_sh)

    plsc.subcore_barrier()

    def gather(chunk, b):
        pltpu.async_copy(table_sh.at[idx_v.at[chunk]], rows[b], sem_g[b])

    def gather_wait(b):
        pltpu.make_async_copy(table_sh.at[idx_v.at[0]], rows[b], sem_g[b]).wait()

    def store(chunk, b):
        dst = out_hbm.at[pl.ds(base + chunk * CHUNK, CHUNK)]
        pltpu.async_copy(rows[b], dst, sem_s[b])

    def store_wait(b):
        dst = out_hbm.at[pl.ds(base, CHUNK)]
        pltpu.make_async_copy(rows[b], dst, sem_s[b]).wait()

    pltpu.sync_copy(idx_hbm.at[wid], idx_v)
    for b in range(NBUF):
        gather(b, b)

    def body(g, carry):
        for b in range(NBUF):
            j = g * NBUF + b
            bp = (b - 1) % NBUF
            if b == 0:
                can_refill = g >= 1
            else:
                can_refill = g <= GROUPS - 2
            @pl.when(can_refill)
            def _refill(bp=bp, chunk=j + NBUF - 1):
                store_wait(bp)
                gather(chunk, bp)
            gather_wait(b)
            store(j, b)
        return carry

    lax.fori_loop(0, GROUPS, body, 0)
    for b in range(NBUF):
        store_wait(b)


def kernel(input_feature, table):
    idx_t = jnp.transpose(input_feature, (1, 2, 0))
    idx = idx_t.reshape(NW, NCHUNK, CHUNK).astype(jnp.int32)
    out = _embed(table, idx)
    return out.reshape(N, L, B, D).transpose(2, 0, 1, 3)
